# baseline (device time: 29134 ns/iter reference)
import jax
import jax.numpy as jnp
from jax import lax
from jax.experimental import pallas as pl
from jax.experimental.pallas import tpu as pltpu

N_DEV = 8


def _stages_local(x, m, ncols, k, j_hi, row_offset):
    j = j_hi
    while j >= 1:
        g = m // (2 * j)
        xr = x.reshape(g, 2 * j, ncols)
        lo, hi = xr[:, :j, :], xr[:, j:, :]
        mn, mx = jnp.minimum(lo, hi), jnp.maximum(lo, hi)
        gidx = lax.broadcasted_iota(jnp.int32, (g, 1, 1), 0)
        off = 0 if k < m else row_offset
        up = ((off + gidx * (2 * j)) & k) == 0
        x = jnp.concatenate(
            [jnp.where(up, mn, mx), jnp.where(up, mx, mn)], axis=1
        ).reshape(m, ncols)
        j //= 2
    return x


def _merge_packed(x, m, ncols, gcols):
    lane = lax.broadcasted_iota(jnp.int32, (m, ncols), 1)
    gi = lane // gcols
    lane3 = lax.broadcasted_iota(jnp.int32, (1, 1, ncols), 2)
    gi3 = lane3 // gcols
    for k in (2 * m, 4 * m, 8 * m):
        j = k // 2
        while j >= m:
            gd = j // m
            shift = gcols * gd
            is_low = (gi & gd) == 0
            partner = jnp.where(
                is_low,
                pltpu.roll(x, ncols - shift, 1),
                pltpu.roll(x, shift, 1),
            )
            up = ((gi * m) & k) == 0
            keep_min = up == is_low
            x = jnp.where(
                keep_min, jnp.minimum(x, partner), jnp.maximum(x, partner)
            )
            j //= 2
        while j >= 1:
            g = m // (2 * j)
            xr = x.reshape(g, 2 * j, ncols)
            lo, hi = xr[:, :j, :], xr[:, j:, :]
            mn, mx = jnp.minimum(lo, hi), jnp.maximum(lo, hi)
            up = ((gi3 * m) & k) == 0
            x = jnp.concatenate(
                [jnp.where(up, mn, mx), jnp.where(up, mx, mn)], axis=1
            ).reshape(m, ncols)
            j //= 2
    return x


def kernel(x):
    m_per, ncols = x.shape
    gcols = ncols // N_DEV

    def body(
        x_ref, out_ref,
        stage1, recv1, stage2, recv2,
        send_sems1, recv_sems1, send_sems2, recv_sems2,
    ):
        my = lax.axis_index("i")

        barrier_sem = pltpu.get_barrier_semaphore()
        for off in range(1, N_DEV):
            pl.semaphore_signal(
                barrier_sem, inc=1,
                device_id=(my ^ off,), device_id_type=pl.DeviceIdType.MESH,
            )
        pl.semaphore_wait(barrier_sem, N_DEV - 1)

        xv = x_ref[...]
        k = 2
        while k <= m_per:
            xv = _stages_local(xv, m_per, ncols, k, k // 2, my * m_per)
            k *= 2

        for d in range(N_DEV):
            stage1[d] = xv[:, d * gcols:(d + 1) * gcols]
        recv1[my] = stage1[my]
        rdmas = []
        for off in range(1, N_DEV):
            tgt = my ^ off
            rdma = pltpu.make_async_remote_copy(
                src_ref=stage1.at[tgt],
                dst_ref=recv1.at[my],
                send_sem=send_sems1.at[off - 1],
                recv_sem=recv_sems1.at[off - 1],
                device_id=(tgt,),
                device_id_type=pl.DeviceIdType.MESH,
            )
            rdma.start()
            rdmas.append(rdma)
        for rdma in rdmas[-(N_DEV - 1):]:
            rdma.wait_recv()

        packed = jnp.concatenate([recv1[s] for s in range(N_DEV)], axis=1)
        merged = _merge_packed(packed, m_per, ncols, gcols)

        for d in range(N_DEV):
            stage2[d] = merged[:, d * gcols:(d + 1) * gcols]
        recv2[my] = stage2[my]
        for off in range(1, N_DEV):
            tgt = my ^ off
            rdma = pltpu.make_async_remote_copy(
                src_ref=stage2.at[tgt],
                dst_ref=recv2.at[my],
                send_sem=send_sems2.at[off - 1],
                recv_sem=recv_sems2.at[off - 1],
                device_id=(tgt,),
                device_id_type=pl.DeviceIdType.MESH,
            )
            rdma.start()
            rdmas.append(rdma)
        for rdma in rdmas[-(N_DEV - 1):]:
            rdma.wait_recv()

        out_ref[...] = jnp.concatenate(
            [recv2[d] for d in range(N_DEV)], axis=1
        )
        for rdma in rdmas:
            rdma.wait_send()

    return pl.pallas_call(
        body,
        out_shape=jax.ShapeDtypeStruct((m_per, ncols), x.dtype),
        in_specs=[pl.BlockSpec(memory_space=pltpu.VMEM)],
        out_specs=pl.BlockSpec(memory_space=pltpu.VMEM),
        scratch_shapes=[
            pltpu.VMEM((N_DEV, m_per, gcols), x.dtype),
            pltpu.VMEM((N_DEV, m_per, gcols), x.dtype),
            pltpu.VMEM((N_DEV, m_per, gcols), x.dtype),
            pltpu.VMEM((N_DEV, m_per, gcols), x.dtype),
            pltpu.SemaphoreType.DMA((N_DEV - 1,)),
            pltpu.SemaphoreType.DMA((N_DEV - 1,)),
            pltpu.SemaphoreType.DMA((N_DEV - 1,)),
            pltpu.SemaphoreType.DMA((N_DEV - 1,)),
        ],
        compiler_params=pltpu.CompilerParams(collective_id=0),
    )(x)


# device time: 16465 ns/iter; 1.7695x vs baseline; 1.7695x over previous
import jax
import jax.numpy as jnp
from jax import lax
from jax.experimental import pallas as pl
from jax.experimental.pallas import tpu as pltpu

N_DEV = 8
M = 256
NCOLS = 128
GCOLS = NCOLS // N_DEV
T = M // 8


def _roll(x, shift):
    return pltpu.roll(x, shift, len(x.shape) - 1)


def _sterm(s_term, ndim):
    if isinstance(s_term, int) or getattr(s_term, "ndim", 0) == 0:
        return s_term
    return s_term.reshape((N_DEV,) + (1,) * (ndim - 1))


def _packed_stage(y, k, j, s_term):
    if j >= M:
        gd = j // M
        ng = N_DEV // (2 * gd)
        yr = y.reshape(ng, 2 * gd, T, NCOLS)
        lo, hi = yr[:, :gd], yr[:, gd:]
        mn, mx = jnp.minimum(lo, hi), jnp.maximum(lo, hi)
        q = lax.broadcasted_iota(jnp.int32, (ng, 1, 1, 1), 0)
        u = lax.broadcasted_iota(jnp.int32, (1, gd, 1, 1), 1)
        up = (((q * 2 * gd + u) * M) & k) == 0
        return jnp.concatenate(
            [jnp.where(up, mn, mx), jnp.where(up, mx, mn)], axis=1
        ).reshape(N_DEV, T, NCOLS)
    if j >= 8:
        jt = j // 8
        gt = T // (2 * jt)
        yr = y.reshape(N_DEV, gt, 2 * jt, NCOLS)
        lo, hi = yr[:, :, :jt], yr[:, :, jt:]
        mn, mx = jnp.minimum(lo, hi), jnp.maximum(lo, hi)
        gidx = lax.broadcasted_iota(jnp.int32, (1, gt, 1, 1), 1)
        st = _sterm(s_term, 4)
        up = ((st + gidx * 2 * jt * 8) & k) == 0
        return jnp.concatenate(
            [jnp.where(up, mn, mx), jnp.where(up, mx, mn)], axis=2
        ).reshape(N_DEV, T, NCOLS)
    shift = GCOLS * j
    lane = lax.broadcasted_iota(jnp.int32, (1, 1, NCOLS), 2)
    u = lane // GCOLS
    t = lax.broadcasted_iota(jnp.int32, (1, T, 1), 1)
    st = _sterm(s_term, 3)
    is_low = (u & j) == 0
    partner = jnp.where(is_low, _roll(y, NCOLS - shift), _roll(y, shift))
    up = ((st + t * 8 + u) & k) == 0
    keep_min = up == is_low
    return jnp.where(keep_min, jnp.minimum(y, partner), jnp.maximum(y, partner))


def _packed_sort_local(y, row0):
    k = 2
    while k <= M:
        j = k // 2
        while j >= 1:
            y = _packed_stage(y, k, j, 0 if k < M else row0)
            j //= 2
        k *= 2
    return y


def _packed_merge(y):
    s = lax.broadcasted_iota(jnp.int32, (N_DEV,), 0) * M
    for k in (2 * M, 4 * M, 8 * M):
        j = k // 2
        while j >= 1:
            y = _packed_stage(y, k, j, s)
            j //= 2
    return y


def _pack(xv):
    xr = xv.reshape(T, 8, NCOLS)
    slabs = []
    for d in range(N_DEV):
        rows = [xr[:, u, d * GCOLS:(d + 1) * GCOLS] for u in range(8)]
        slabs.append(jnp.concatenate(rows, axis=1)[None])
    return jnp.concatenate(slabs, axis=0)


def _unpack(slabs):
    planes = []
    for u in range(8):
        row_u = jnp.concatenate(
            [s[:, u * GCOLS:(u + 1) * GCOLS] for s in slabs], axis=1
        )
        planes.append(row_u[:, None, :])
    return jnp.concatenate(planes, axis=1).reshape(M, NCOLS)


def kernel(x):
    assert x.shape == (M, NCOLS)

    def body(
        x_ref, out_ref,
        stage1, recv1, stage2, recv2,
        send_sems1, recv_sems1, send_sems2, recv_sems2,
    ):
        my = lax.axis_index("i")

        barrier_sem = pltpu.get_barrier_semaphore()
        for off in range(1, N_DEV):
            pl.semaphore_signal(
                barrier_sem, inc=1,
                device_id=(my ^ off,), device_id_type=pl.DeviceIdType.MESH,
            )
        pl.semaphore_wait(barrier_sem, N_DEV - 1)

        y = _packed_sort_local(_pack(x_ref[...]), my * M)
        stage1[...] = y

        recv1[my] = stage1[my]
        rdmas = []
        for off in range(1, N_DEV):
            tgt = my ^ off
            rdma = pltpu.make_async_remote_copy(
                src_ref=stage1.at[tgt],
                dst_ref=recv1.at[my],
                send_sem=send_sems1.at[off - 1],
                recv_sem=recv_sems1.at[off - 1],
                device_id=(tgt,),
                device_id_type=pl.DeviceIdType.MESH,
            )
            rdma.start()
            rdmas.append(rdma)
        for rdma in rdmas[-(N_DEV - 1):]:
            rdma.wait_recv()

        stage2[...] = _packed_merge(recv1[...])

        recv2[my] = stage2[my]
        for off in range(1, N_DEV):
            tgt = my ^ off
            rdma = pltpu.make_async_remote_copy(
                src_ref=stage2.at[tgt],
                dst_ref=recv2.at[my],
                send_sem=send_sems2.at[off - 1],
                recv_sem=recv_sems2.at[off - 1],
                device_id=(tgt,),
                device_id_type=pl.DeviceIdType.MESH,
            )
            rdma.start()
            rdmas.append(rdma)
        for rdma in rdmas[-(N_DEV - 1):]:
            rdma.wait_recv()

        out_ref[...] = _unpack([recv2[d] for d in range(N_DEV)])
        for rdma in rdmas:
            rdma.wait_send()

    return pl.pallas_call(
        body,
        out_shape=jax.ShapeDtypeStruct((M, NCOLS), x.dtype),
        in_specs=[pl.BlockSpec(memory_space=pltpu.VMEM)],
        out_specs=pl.BlockSpec(memory_space=pltpu.VMEM),
        scratch_shapes=[
            pltpu.VMEM((N_DEV, T, NCOLS), x.dtype),
            pltpu.VMEM((N_DEV, T, NCOLS), x.dtype),
            pltpu.VMEM((N_DEV, T, NCOLS), x.dtype),
            pltpu.VMEM((N_DEV, T, NCOLS), x.dtype),
            pltpu.SemaphoreType.DMA((N_DEV - 1,)),
            pltpu.SemaphoreType.DMA((N_DEV - 1,)),
            pltpu.SemaphoreType.DMA((N_DEV - 1,)),
            pltpu.SemaphoreType.DMA((N_DEV - 1,)),
        ],
        compiler_params=pltpu.CompilerParams(collective_id=0),
    )(x)
